# baseline (speedup 1.0000x reference)
import functools

import jax
import jax.numpy as jnp
from jax.experimental import pallas as pl
from jax.experimental.pallas import tpu as pltpu

BLK1 = 256
BLKA_T = 512
BLKA = 256
BLKB = 256
BLK3 = 256


def _exp_kernel(sims_ref, s0b_ref, s0bt_ref, r_ref):
    b = pl.program_id(0)
    e = jnp.exp(sims_ref[...] * jnp.float32(50.0))
    eb = e.astype(jnp.bfloat16)
    s0b_ref[...] = eb
    s0bt_ref[...] = eb.T
    rowsum = jnp.sum(e, axis=1, keepdims=True)
    r_ref[pl.ds(b * BLK1, BLK1), :] = jnp.float32(1.0) / rowsum


def _bf16_steps_kernel(r1_ref, s0bt_ref, s0b_ref, r_ref, c_ref):
    s = pl.program_id(0)
    b = pl.program_id(1)

    @pl.when((s == 0) & (b == 0))
    def _():
        r_ref[...] = r1_ref[...]

    @pl.when(s % 2 == 0)
    def _():
        part = jax.lax.dot_general(
            s0bt_ref[...], r_ref[...].astype(jnp.bfloat16),
            (((1,), (0,)), ((), ())),
            preferred_element_type=jnp.float32,
        )
        c_ref[pl.ds(b * BLKA_T, BLKA_T), :] = jnp.float32(1.0) / part

    @pl.when(s % 2 == 1)
    def _():
        part = jax.lax.dot_general(
            s0b_ref[...], c_ref[...].astype(jnp.bfloat16),
            (((1,), (0,)), ((), ())),
            preferred_element_type=jnp.float32,
        )
        r_ref[pl.ds(b * BLKA, BLKA), :] = jnp.float32(1.0) / part


def _f32_steps_kernel(c4_ref, sims_ref, r_ref, c_ref, c4row_ref, nblk):
    s = pl.program_id(0)
    b = pl.program_id(1)

    @pl.when((s == 0) & (b == 0))
    def _():
        c4row_ref[...] = c4_ref[...].T

    blk = jnp.exp(sims_ref[...] * jnp.float32(50.0))

    @pl.when(s == 0)
    def _():
        rowsum = jnp.sum(blk * c4row_ref[...], axis=1, keepdims=True)
        r_ref[pl.ds(b * BLKB, BLKB), :] = jnp.float32(1.0) / rowsum

    @pl.when(s == 1)
    def _():
        rblk = r_ref[pl.ds(b * BLKB, BLKB), :]
        part = jnp.sum(blk * rblk, axis=0, keepdims=True)

        @pl.when(b == 0)
        def _():
            c_ref[...] = part

        @pl.when(b != 0)
        def _():
            c_ref[...] = c_ref[...] + part

        @pl.when(b == nblk - 1)
        def _():
            c_ref[...] = jnp.float32(1.0) / c_ref[...]


def _finalize_kernel(sims_ref, r_ref, c_ref, s_ref, col_ref):
    b = pl.program_id(0)
    blk = jnp.exp(sims_ref[...] * jnp.float32(50.0))
    rblk = r_ref[pl.ds(b * BLK3, BLK3), :]
    sblk = blk * rblk * c_ref[...]
    s_ref[...] = sblk
    col_ref[pl.ds(b * BLK3, BLK3), :] = jnp.argmax(
        sblk, axis=1, keepdims=True
    ).astype(jnp.int32)


def kernel(sims, batch_size):
    num_row, num_col = sims.shape

    s0b, s0bt, r1 = pl.pallas_call(
        _exp_kernel,
        grid=(num_row // BLK1,),
        in_specs=[pl.BlockSpec((BLK1, num_col), lambda b: (b, 0))],
        out_specs=[
            pl.BlockSpec((BLK1, num_col), lambda b: (b, 0)),
            pl.BlockSpec((num_col, BLK1), lambda b: (0, b)),
            pl.BlockSpec((num_row, 1), lambda b: (0, 0)),
        ],
        out_shape=[
            jax.ShapeDtypeStruct((num_row, num_col), jnp.bfloat16),
            jax.ShapeDtypeStruct((num_col, num_row), jnp.bfloat16),
            jax.ShapeDtypeStruct((num_row, 1), jnp.float32),
        ],
    )(sims)

    nblka_t = num_col // BLKA_T
    nblka = num_row // BLKA
    assert nblka_t == nblka
    r4, c4 = pl.pallas_call(
        _bf16_steps_kernel,
        grid=(7, nblka),
        in_specs=[
            pl.BlockSpec((num_row, 1), lambda s, b: (0, 0)),
            pl.BlockSpec(
                (BLKA_T, num_row),
                lambda s, b: (jnp.where(s % 2 == 0, b, nblka - 1), 0),
            ),
            pl.BlockSpec(
                (BLKA, num_col),
                lambda s, b: (jnp.where(s % 2 == 1, b, nblka - 1), 0),
            ),
        ],
        out_specs=[
            pl.BlockSpec((num_row, 1), lambda s, b: (0, 0)),
            pl.BlockSpec((num_col, 1), lambda s, b: (0, 0)),
        ],
        out_shape=[
            jax.ShapeDtypeStruct((num_row, 1), jnp.float32),
            jax.ShapeDtypeStruct((num_col, 1), jnp.float32),
        ],
    )(r1, s0bt, s0b)

    nblkb = num_row // BLKB
    r5, c5 = pl.pallas_call(
        functools.partial(_f32_steps_kernel, nblk=nblkb),
        grid=(2, nblkb),
        in_specs=[
            pl.BlockSpec((num_col, 1), lambda s, b: (0, 0)),
            pl.BlockSpec((BLKB, num_col), lambda s, b: (b, 0)),
        ],
        out_specs=[
            pl.BlockSpec((num_row, 1), lambda s, b: (0, 0)),
            pl.BlockSpec((1, num_col), lambda s, b: (0, 0)),
        ],
        out_shape=[
            jax.ShapeDtypeStruct((num_row, 1), jnp.float32),
            jax.ShapeDtypeStruct((1, num_col), jnp.float32),
        ],
        scratch_shapes=[pltpu.VMEM((1, num_col), jnp.float32)],
    )(c4, sims)

    s, col = pl.pallas_call(
        _finalize_kernel,
        grid=(num_row // BLK3,),
        in_specs=[
            pl.BlockSpec((BLK3, num_col), lambda b: (b, 0)),
            pl.BlockSpec((num_row, 1), lambda b: (0, 0)),
            pl.BlockSpec((1, num_col), lambda b: (0, 0)),
        ],
        out_specs=[
            pl.BlockSpec((BLK3, num_col), lambda b: (b, 0)),
            pl.BlockSpec((num_row, 1), lambda b: (0, 0)),
        ],
        out_shape=[
            jax.ShapeDtypeStruct((num_row, num_col), jnp.float32),
            jax.ShapeDtypeStruct((num_row, 1), jnp.int32),
        ],
    )(sims, r5, c5)

    row = jnp.arange(num_row, dtype=jnp.int32)
    indices = jnp.stack((row, col.reshape(num_row)), axis=0)
    values = jnp.ones((num_row,), dtype=jnp.float32)
    return (s, indices, values)


# speedup vs baseline: 1.0112x; 1.0112x over previous
import functools

import jax
import jax.numpy as jnp
from jax.experimental import pallas as pl
from jax.experimental.pallas import tpu as pltpu

BLK1 = 256
BLKA_T = 1024
BLKA = 512
BLKB = 256
BLK3 = 256


def _exp_kernel(sims_ref, s0b_ref, s0bt_ref, r_ref):
    b = pl.program_id(0)
    e = jnp.exp(sims_ref[...] * jnp.float32(50.0))
    eb = e.astype(jnp.bfloat16)
    s0b_ref[...] = eb
    s0bt_ref[...] = eb.T
    rowsum = jnp.sum(e, axis=1, keepdims=True)
    r_ref[pl.ds(b * BLK1, BLK1), :] = jnp.float32(1.0) / rowsum


def _bf16_matvec_kernel(mat_ref, vec_ref, out_ref, blk_rows):
    b = pl.program_id(0)
    part = jax.lax.dot_general(
        mat_ref[...], vec_ref[...].astype(jnp.bfloat16),
        (((1,), (0,)), ((), ())),
        preferred_element_type=jnp.float32,
    )
    out_ref[pl.ds(b * blk_rows, blk_rows), :] = jnp.float32(1.0) / part


def _bf16_matvec(mat, vec, blk_rows):
    n, k = mat.shape
    return pl.pallas_call(
        functools.partial(_bf16_matvec_kernel, blk_rows=blk_rows),
        grid=(n // blk_rows,),
        in_specs=[
            pl.BlockSpec((blk_rows, k), lambda b: (b, 0)),
            pl.BlockSpec((k, 1), lambda b: (0, 0)),
        ],
        out_specs=pl.BlockSpec((n, 1), lambda b: (0, 0)),
        out_shape=jax.ShapeDtypeStruct((n, 1), jnp.float32),
    )(mat, vec)


def _f32_steps_kernel(c4_ref, sims_ref, r_ref, c_ref, c4row_ref, nblk):
    s = pl.program_id(0)
    b = pl.program_id(1)

    @pl.when((s == 0) & (b == 0))
    def _():
        c4row_ref[...] = c4_ref[...].T

    blk = jnp.exp(sims_ref[...] * jnp.float32(50.0))

    @pl.when(s == 0)
    def _():
        rowsum = jnp.sum(blk * c4row_ref[...], axis=1, keepdims=True)
        r_ref[pl.ds(b * BLKB, BLKB), :] = jnp.float32(1.0) / rowsum

    @pl.when(s == 1)
    def _():
        rblk = r_ref[pl.ds(b * BLKB, BLKB), :]
        part = jnp.sum(blk * rblk, axis=0, keepdims=True)

        @pl.when(b == 0)
        def _():
            c_ref[...] = part

        @pl.when(b != 0)
        def _():
            c_ref[...] = c_ref[...] + part

        @pl.when(b == nblk - 1)
        def _():
            c_ref[...] = jnp.float32(1.0) / c_ref[...]


def _finalize_kernel(sims_ref, r_ref, c_ref, s_ref, col_ref):
    b = pl.program_id(0)
    blk = jnp.exp(sims_ref[...] * jnp.float32(50.0))
    rblk = r_ref[pl.ds(b * BLK3, BLK3), :]
    sblk = blk * rblk * c_ref[...]
    s_ref[...] = sblk
    col_ref[pl.ds(b * BLK3, BLK3), :] = jnp.argmax(
        sblk, axis=1, keepdims=True
    ).astype(jnp.int32)


def kernel(sims, batch_size):
    num_row, num_col = sims.shape

    s0b, s0bt, r1 = pl.pallas_call(
        _exp_kernel,
        grid=(num_row // BLK1,),
        in_specs=[pl.BlockSpec((BLK1, num_col), lambda b: (b, 0))],
        out_specs=[
            pl.BlockSpec((BLK1, num_col), lambda b: (b, 0)),
            pl.BlockSpec((num_col, BLK1), lambda b: (0, b)),
            pl.BlockSpec((num_row, 1), lambda b: (0, 0)),
        ],
        out_shape=[
            jax.ShapeDtypeStruct((num_row, num_col), jnp.bfloat16),
            jax.ShapeDtypeStruct((num_col, num_row), jnp.bfloat16),
            jax.ShapeDtypeStruct((num_row, 1), jnp.float32),
        ],
    )(sims)

    r = r1
    c = None
    for step in range(7):
        if step % 2 == 0:
            c = _bf16_matvec(s0bt, r, BLKA_T)
        else:
            r = _bf16_matvec(s0b, c, BLKA)
    c4 = c

    nblkb = num_row // BLKB
    r5, c5 = pl.pallas_call(
        functools.partial(_f32_steps_kernel, nblk=nblkb),
        grid=(2, nblkb),
        in_specs=[
            pl.BlockSpec((num_col, 1), lambda s, b: (0, 0)),
            pl.BlockSpec((BLKB, num_col), lambda s, b: (b, 0)),
        ],
        out_specs=[
            pl.BlockSpec((num_row, 1), lambda s, b: (0, 0)),
            pl.BlockSpec((1, num_col), lambda s, b: (0, 0)),
        ],
        out_shape=[
            jax.ShapeDtypeStruct((num_row, 1), jnp.float32),
            jax.ShapeDtypeStruct((1, num_col), jnp.float32),
        ],
        scratch_shapes=[pltpu.VMEM((1, num_col), jnp.float32)],
    )(c4, sims)

    s, col = pl.pallas_call(
        _finalize_kernel,
        grid=(num_row // BLK3,),
        in_specs=[
            pl.BlockSpec((BLK3, num_col), lambda b: (b, 0)),
            pl.BlockSpec((num_row, 1), lambda b: (0, 0)),
            pl.BlockSpec((1, num_col), lambda b: (0, 0)),
        ],
        out_specs=[
            pl.BlockSpec((BLK3, num_col), lambda b: (b, 0)),
            pl.BlockSpec((num_row, 1), lambda b: (0, 0)),
        ],
        out_shape=[
            jax.ShapeDtypeStruct((num_row, num_col), jnp.float32),
            jax.ShapeDtypeStruct((num_row, 1), jnp.int32),
        ],
    )(sims, r5, c5)

    row = jnp.arange(num_row, dtype=jnp.int32)
    indices = jnp.stack((row, col.reshape(num_row)), axis=0)
    values = jnp.ones((num_row,), dtype=jnp.float32)
    return (s, indices, values)
